# baseline (speedup 1.0000x reference)
import functools

import jax
import jax.numpy as jnp
from jax import lax
from jax.experimental import pallas as pl
from jax.experimental.pallas import tpu as pltpu
from jax.experimental.pallas import tpu_sc as plsc

N = 10000
E = 320000
D = 128
NC = 2
NS = 16
NW = NC * NS
EPW = E // NW
K = 80
NCHUNK = EPW // K
SC_C = 25
SC_S = NCHUNK // SC_C
RPS = 624
RTAIL = N - RPS * NS


def _mm1_body(x_ref, w_ref, b_ref, o_ref):
    o_ref[...] = jnp.maximum(
        jnp.dot(x_ref[...], w_ref[...], preferred_element_type=jnp.float32)
        + b_ref[...], 0.0)


def _post_body(feat_ref, num_ref, den_ref, deg_ref, w1_ref, w2_ref, b_ref,
               o_ref):
    num = num_ref[0] + num_ref[1]
    den = jnp.sum(den_ref[...], axis=0)[:, None]
    deg = jnp.sum(deg_ref[...], axis=0)[:, None]
    safe_den = jnp.where(den > 0, den, 1.0)
    agg = jnp.where(den > 0, num / safe_den, 0.0)
    rst = jnp.dot(feat_ref[...], w1_ref[...], preferred_element_type=jnp.float32)
    rst = rst + jnp.dot(agg, w2_ref[...], preferred_element_type=jnp.float32)
    rst = jnp.maximum(rst + b_ref[...], 0.0)
    rst = rst * (1.0 / jnp.maximum(deg, 1.0))
    denom = jnp.sqrt(jnp.sum(rst * rst, axis=1, keepdims=True))
    safe_denom = jnp.where(denom > 0, denom, 1.0)
    o_ref[...] = jnp.where(denom > 0, rst / safe_denom, 0.0)


def _sc_body(h_hbm, src_hbm, dst_hbm, ew_hbm, z128_hbm,
             num_out, den_out, deg_out,
             src_v, dst_v, ew_v, rows_v, den_v, deg_v, acc_sh,
             sem_g0, sem_g1, sem_i0):
    cid = lax.axis_index("c")
    sid = lax.axis_index("s")
    wid = cid * NS + sid
    sem_g = (sem_g0, sem_g1)

    row0 = sid * RPS
    pltpu.sync_copy(z128_hbm.at[pl.ds(row0, RPS)], acc_sh.at[pl.ds(row0, RPS)])

    @pl.when(sid == NS - 1)
    def _init_tail():
        tl = pl.ds(N - RTAIL, RTAIL)
        pltpu.sync_copy(z128_hbm.at[tl], acc_sh.at[tl])

    def _zdd(i, c):
        sl = pl.ds(i * 16, 16)
        den_v[sl] = jnp.zeros((16,), jnp.float32)
        deg_v[sl] = jnp.zeros((16,), jnp.float32)
        return c
    lax.fori_loop(0, N // 16, _zdd, 0)

    plsc.subcore_barrier()

    ones16 = jnp.ones((16,), jnp.float32)

    def _compute_scatter(c):
        def _blk(t, cr):
            w16 = ew_v[c, pl.ds(t * 16, 16)]
            for l in range(16):
                wk = jnp.full((16,), w16[l], jnp.float32)
                i = t * 16 + l
                for g in range(D // 16):
                    sl = pl.ds(g * 16, 16)
                    rows_v[i, sl] = rows_v[i, sl] * wk
            return cr
        lax.fori_loop(0, K // 16, _blk, 0)

        for t in range(K // 16):
            sl = pl.ds(t * 16, 16)
            iv = dst_v[c, sl]
            plsc.addupdate_scatter(den_v, [iv], ew_v[c, sl])
            plsc.addupdate_scatter(deg_v, [iv], ones16)

        pltpu.sync_copy(rows_v, acc_sh.at[dst_v.at[c]], add=True)

    def _super(s, carry):
        pltpu.async_copy(src_hbm.at[wid, s], src_v, sem_i0)
        pltpu.async_copy(dst_hbm.at[wid, s], dst_v, sem_i0)
        pltpu.async_copy(ew_hbm.at[wid, s], ew_v, sem_i0)
        pltpu.make_async_copy(src_hbm.at[wid, s], src_v, sem_i0).wait()
        pltpu.make_async_copy(dst_hbm.at[wid, s], dst_v, sem_i0).wait()
        pltpu.make_async_copy(ew_hbm.at[wid, s], ew_v, sem_i0).wait()

        def _chunk(c, cr):
            pltpu.async_copy(h_hbm.at[src_v.at[c]], rows_v, sem_g[0]).wait()
            _compute_scatter(c)
            return cr
        lax.fori_loop(0, SC_C, _chunk, 0)
        return carry

    lax.fori_loop(0, SC_S, _super, 0)

    plsc.subcore_barrier()

    pltpu.sync_copy(acc_sh.at[pl.ds(row0, RPS)],
                    num_out.at[cid, pl.ds(row0, RPS)])

    @pl.when(sid == NS - 1)
    def _out_tail():
        tl = pl.ds(N - RTAIL, RTAIL)
        pltpu.sync_copy(acc_sh.at[tl], num_out.at[cid, tl])

    pltpu.sync_copy(den_v, den_out.at[wid])
    pltpu.sync_copy(deg_v, deg_out.at[wid])


@functools.lru_cache(maxsize=None)
def _get_sc_call():
    return pl.kernel(
        _sc_body,
        out_type=[jax.ShapeDtypeStruct((NC, N, D), jnp.float32),
                  jax.ShapeDtypeStruct((NW, N), jnp.float32),
                  jax.ShapeDtypeStruct((NW, N), jnp.float32)],
        mesh=plsc.VectorSubcoreMesh(core_axis_name="c", subcore_axis_name="s",
                                    num_cores=NC, num_subcores=NS),
        compiler_params=pltpu.CompilerParams(needs_layout_passes=False),
        scratch_types=[
            pltpu.VMEM((SC_C, K), jnp.int32),
            pltpu.VMEM((SC_C, K), jnp.int32),
            pltpu.VMEM((SC_C, K), jnp.float32),
            pltpu.VMEM((K, D), jnp.float32),
            pltpu.VMEM((N,), jnp.float32),
            pltpu.VMEM((N,), jnp.float32),
            pltpu.VMEM_SHARED((N, D), jnp.float32),
            pltpu.SemaphoreType.DMA,
            pltpu.SemaphoreType.DMA,
            pltpu.SemaphoreType.DMA,
        ],
    )


def kernel(feat, edge_index, edge_w, Q_w, Q_b, W_w, W_b):
    feat = feat.astype(jnp.float32)
    src = edge_index[0].astype(jnp.int32)
    dst = edge_index[1].astype(jnp.int32)
    ew = edge_w.astype(jnp.float32)

    h = pl.pallas_call(
        _mm1_body,
        out_shape=jax.ShapeDtypeStruct((N, D), jnp.float32),
    )(feat, Q_w, Q_b.reshape(1, D))

    z128 = jnp.zeros((N, D), jnp.float32)
    num_p, den_p, deg_p = _get_sc_call()(
        h, src.reshape(NW, SC_S, SC_C, K), dst.reshape(NW, SC_S, SC_C, K),
        ew.reshape(NW, SC_S, SC_C, K), z128)

    rst = pl.pallas_call(
        _post_body,
        out_shape=jax.ShapeDtypeStruct((N, D), jnp.float32),
    )(feat, num_p, den_p, deg_p, W_w[:D], W_w[D:], W_b.reshape(1, D))
    return rst


# speedup vs baseline: 1.0930x; 1.0930x over previous
import functools

import jax
import jax.numpy as jnp
from jax import lax
from jax.experimental import pallas as pl
from jax.experimental.pallas import tpu as pltpu
from jax.experimental.pallas import tpu_sc as plsc

N = 10000
E = 320000
D = 128
NC = 2
NS = 16
NW = NC * NS
EPW = E // NW
K = 80
NCHUNK = EPW // K
SC_C = 5
SC_S = NCHUNK // SC_C
RPS = 624
RTAIL = N - RPS * NS


def _mm1_body(x_ref, w_ref, b_ref, o_ref):
    o_ref[...] = jnp.maximum(
        jnp.dot(x_ref[...], w_ref[...], preferred_element_type=jnp.float32)
        + b_ref[...], 0.0)


def _post_body(feat_ref, num_ref, den_ref, deg_ref, w1_ref, w2_ref, b_ref,
               o_ref):
    num = num_ref[0] + num_ref[1]
    den = jnp.sum(den_ref[...], axis=0)[:, None]
    deg = jnp.sum(deg_ref[...], axis=0)[:, None]
    safe_den = jnp.where(den > 0, den, 1.0)
    agg = jnp.where(den > 0, num / safe_den, 0.0)
    rst = jnp.dot(feat_ref[...], w1_ref[...], preferred_element_type=jnp.float32)
    rst = rst + jnp.dot(agg, w2_ref[...], preferred_element_type=jnp.float32)
    rst = jnp.maximum(rst + b_ref[...], 0.0)
    rst = rst * (1.0 / jnp.maximum(deg, 1.0))
    denom = jnp.sqrt(jnp.sum(rst * rst, axis=1, keepdims=True))
    safe_denom = jnp.where(denom > 0, denom, 1.0)
    o_ref[...] = jnp.where(denom > 0, rst / safe_denom, 0.0)


def _sc_body(h_hbm, edata_hbm, z128_hbm,
             num_out, den_out, deg_out,
             idx_v, rows_v, den_v, deg_v, acc_sh,
             sem_g0, sem_g1, sem_i0, sem_s0):
    cid = lax.axis_index("c")
    sid = lax.axis_index("s")
    wid = cid * NS + sid
    sem_g = (sem_g0, sem_g1)

    row0 = sid * RPS
    pltpu.sync_copy(z128_hbm.at[pl.ds(row0, RPS)], acc_sh.at[pl.ds(row0, RPS)])

    @pl.when(sid == NS - 1)
    def _init_tail():
        tl = pl.ds(N - RTAIL, RTAIL)
        pltpu.sync_copy(z128_hbm.at[tl], acc_sh.at[tl])

    def _zdd(i, c):
        sl = pl.ds(i * 16, 16)
        den_v[sl] = jnp.zeros((16,), jnp.float32)
        deg_v[sl] = jnp.zeros((16,), jnp.float32)
        return c
    lax.fori_loop(0, N // 16, _zdd, 0)

    plsc.subcore_barrier()

    ones16 = jnp.ones((16,), jnp.float32)

    def _compute(c, b):
        def _blk(t, cr):
            w16 = plsc.bitcast(idx_v[3 * c + 2, pl.ds(t * 16, 16)], jnp.float32)
            for l in range(16):
                wk = jnp.full((16,), w16[l], jnp.float32)
                i = t * 16 + l
                for g in range(D // 16):
                    sl = pl.ds(g * 16, 16)
                    rows_v[b, i, sl] = rows_v[b, i, sl] * wk
            return cr
        lax.fori_loop(0, K // 16, _blk, 0)

        for t in range(K // 16):
            sl = pl.ds(t * 16, 16)
            iv = idx_v[3 * c + 1, sl]
            plsc.addupdate_scatter(den_v, [iv],
                                   plsc.bitcast(idx_v[3 * c + 2, sl],
                                                jnp.float32))
            plsc.addupdate_scatter(deg_v, [iv], ones16)

    def _super(s, carry):
        pltpu.async_copy(edata_hbm.at[wid, s], idx_v, sem_i0).wait()

        def _pairc(p, cr):
            c0 = 2 * p
            c1 = c0 + 1
            d0 = pltpu.async_copy(h_hbm.at[idx_v.at[3 * c0]], rows_v.at[0],
                                  sem_g[0])
            d1 = pltpu.async_copy(h_hbm.at[idx_v.at[3 * c1]], rows_v.at[1],
                                  sem_g[1])
            d0.wait()
            _compute(c0, 0)
            s0 = pltpu.async_copy(rows_v.at[0], acc_sh.at[idx_v.at[3 * c0 + 1]],
                                  sem_s0, add=True)
            d1.wait()
            _compute(c1, 1)
            s0.wait()
            pltpu.sync_copy(rows_v.at[1], acc_sh.at[idx_v.at[3 * c1 + 1]], add=True)
            return cr
        lax.fori_loop(0, SC_C // 2, _pairc, 0)

        c = SC_C - 1
        pltpu.async_copy(h_hbm.at[idx_v.at[3 * c]], rows_v.at[0],
                         sem_g[0]).wait()
        _compute(c, 0)
        pltpu.sync_copy(rows_v.at[0], acc_sh.at[idx_v.at[3 * c + 1]], add=True)
        return carry

    lax.fori_loop(0, SC_S, _super, 0)

    plsc.subcore_barrier()

    pltpu.sync_copy(acc_sh.at[pl.ds(row0, RPS)],
                    num_out.at[cid, pl.ds(row0, RPS)])

    @pl.when(sid == NS - 1)
    def _out_tail():
        tl = pl.ds(N - RTAIL, RTAIL)
        pltpu.sync_copy(acc_sh.at[tl], num_out.at[cid, tl])

    pltpu.sync_copy(den_v, den_out.at[wid])
    pltpu.sync_copy(deg_v, deg_out.at[wid])


@functools.lru_cache(maxsize=None)
def _get_sc_call():
    return pl.kernel(
        _sc_body,
        out_type=[jax.ShapeDtypeStruct((NC, N, D), jnp.float32),
                  jax.ShapeDtypeStruct((NW, N), jnp.float32),
                  jax.ShapeDtypeStruct((NW, N), jnp.float32)],
        mesh=plsc.VectorSubcoreMesh(core_axis_name="c", subcore_axis_name="s",
                                    num_cores=NC, num_subcores=NS),
        compiler_params=pltpu.CompilerParams(needs_layout_passes=False),
        scratch_types=[
            pltpu.VMEM((SC_C * 3, K), jnp.int32),
            pltpu.VMEM((2, K, D), jnp.float32),
            pltpu.VMEM((N,), jnp.float32),
            pltpu.VMEM((N,), jnp.float32),
            pltpu.VMEM_SHARED((N, D), jnp.float32),
            pltpu.SemaphoreType.DMA,
            pltpu.SemaphoreType.DMA,
            pltpu.SemaphoreType.DMA,
            pltpu.SemaphoreType.DMA,
        ],
    )


def kernel(feat, edge_index, edge_w, Q_w, Q_b, W_w, W_b):
    feat = feat.astype(jnp.float32)
    src = edge_index[0].astype(jnp.int32)
    dst = edge_index[1].astype(jnp.int32)
    ew = edge_w.astype(jnp.float32)

    h = pl.pallas_call(
        _mm1_body,
        out_shape=jax.ShapeDtypeStruct((N, D), jnp.float32),
    )(feat, Q_w, Q_b.reshape(1, D))

    z128 = jnp.zeros((N, D), jnp.float32)
    edata = jnp.stack(
        [src.reshape(NW, SC_S, SC_C, K), dst.reshape(NW, SC_S, SC_C, K),
         lax.bitcast_convert_type(ew, jnp.int32).reshape(NW, SC_S, SC_C, K)],
        axis=3).reshape(NW, SC_S, SC_C * 3, K)
    num_p, den_p, deg_p = _get_sc_call()(h, edata, z128)

    rst = pl.pallas_call(
        _post_body,
        out_shape=jax.ShapeDtypeStruct((N, D), jnp.float32),
    )(feat, num_p, den_p, deg_p, W_w[:D], W_w[D:], W_b.reshape(1, D))
    return rst
